# baseline (speedup 1.0000x reference)
import functools

import jax
import jax.numpy as jnp
from jax import lax
from jax.experimental import pallas as pl
from jax.experimental.pallas import tpu as pltpu
from jax.experimental.pallas import tpu_sc as plsc

D = 128
D2 = 2 * D
HID = 2 * D2
H4 = 4 * HID
B = 1024
FEW = 5
STEPS = 4


_NW = 32
_NQ = 2 * B
_QPW = _NQ // _NW
_CH = 8
_NST = _QPW // _CH
_NS = 16


def _sc_gather_body(table_hbm, idxq_hbm, idxs_hbm, outq_hbm, outs_hbm,
                    idx_v, rows_v, idxs_v, rows_s, sem, sem_s):
    wid = lax.axis_index("s") * 2 + lax.axis_index("c")
    base = wid * _QPW
    pltpu.sync_copy(idxq_hbm.at[pl.ds(base, _QPW)], idx_v)
    copies = [
        pltpu.async_copy(
            table_hbm.at[idx_v.at[pl.ds(j * _CH, _CH)]],
            rows_v.at[pl.ds(j * _CH, _CH), :], sem)
        for j in range(_NST)
    ]

    @pl.when(wid == 0)
    def _():
        pltpu.sync_copy(idxs_hbm, idxs_v)
        pltpu.async_copy(table_hbm.at[idxs_v], rows_s, sem_s).wait()
        pltpu.sync_copy(rows_s, outs_hbm)

    for c in copies:
        c.wait()
    pltpu.sync_copy(rows_v, outq_hbm.at[pl.ds(base, _QPW)])


@functools.cache
def _make_sc_gather():
    return pl.kernel(
        _sc_gather_body,
        out_type=(
            jax.ShapeDtypeStruct((_NQ, D), jnp.float32),
            jax.ShapeDtypeStruct((_NS, D), jnp.float32),
        ),
        mesh=plsc.VectorSubcoreMesh(core_axis_name="c", subcore_axis_name="s"),
        scratch_types=[
            pltpu.VMEM((_QPW,), jnp.int32),
            pltpu.VMEM((_QPW, D), jnp.float32),
            pltpu.VMEM((_NS,), jnp.int32),
            pltpu.VMEM((_NS, D), jnp.float32),
            pltpu.SemaphoreType.DMA,
            pltpu.SemaphoreType.DMA,
        ],
    )


def _sc_gather(table, idx_q, idx_s):
    return _make_sc_gather()(table, idx_q, idx_s)


def _sigmoid(x):
    return 0.5 * jnp.tanh(0.5 * x) + 0.5


def _encode(x, W1, b1, W2, b2, ln_g, ln_b):
    h = jnp.maximum(jnp.dot(x, W1, preferred_element_type=jnp.float32) + b1, 0.0)
    h = jnp.dot(h, W2, preferred_element_type=jnp.float32) + b2
    y = h + x
    mu = jnp.mean(y, axis=-1, keepdims=True)
    var = jnp.mean((y - mu) * (y - mu), axis=-1, keepdims=True)
    return ln_g * (y - mu) * lax.rsqrt(var + 1e-5) + ln_b


def _dot_nt(x, w):
    return lax.dot_general(x, w, (((1,), (1,)), ((), ())),
                           preferred_element_type=jnp.float32)


_G4 = 4 * D2


def _tc_body(q_ref, s_ref, W1_ref, b1_ref, W2_ref, b2_ref, lng_ref, lnb_ref,
             Wih_hbm, Whh_hbm, b4_ref, out_ref,
             wih_v, whh_v, sem_ih, sem_hh):
    cps_ih = [pltpu.async_copy(Wih_hbm.at[pl.ds(k * HID, D2)],
                               wih_v.at[pl.ds(k * D2, D2)], sem_ih)
              for k in range(4)]
    cps_hh = [pltpu.async_copy(Whh_hbm.at[pl.ds(k * HID, D2)],
                               whh_v.at[pl.ds(k * D2, D2)], sem_hh)
              for k in range(4)]

    W1 = W1_ref[...]
    b1 = b1_ref[...]
    W2 = W2_ref[...]
    b2 = b2_ref[...]
    ln_g = lng_ref[...]
    ln_b = lnb_ref[...]

    s_g = _encode(s_ref[...], W1, b1, W2, b2, ln_g, ln_b)
    row = lax.broadcasted_iota(jnp.int32, (8, 1), 0)
    s_g = jnp.where(row < FEW, s_g, 0.0)
    s_mean = jnp.sum(s_g, axis=0, keepdims=True) * (1.0 / FEW)

    q_g = _encode(q_ref[...], W1, b1, W2, b2, ln_g, ln_b)

    for cp in cps_ih:
        cp.wait()
    a = _dot_nt(q_g, wih_v[...]) + b4_ref[...]

    for cp in cps_hh:
        cp.wait()
    Whh_h = whh_v[:, :D2]
    Whh_r = whh_v[:, D2:]
    r_row = _dot_nt(s_mean, Whh_r)

    c = None
    h = None
    gates = a
    for step in range(STEPS):
        if step > 0:
            gates = a + r_row + _dot_nt(h, Whh_h)
        i = _sigmoid(gates[:, :D2])
        f = _sigmoid(gates[:, D2:2 * D2])
        g = jnp.tanh(gates[:, 2 * D2:3 * D2])
        o = _sigmoid(gates[:, 3 * D2:])
        c = f * c + i * g if step > 0 else i * g
        h = q_g + o * jnp.tanh(c)

    out_ref[...] = jnp.sum(h * s_mean, axis=1, keepdims=True)


@jax.jit
def _tc_dense(q, s8, W1, b1, W2, b2, ln_g, ln_b, W_ih, W_hh, b4):
    full = lambda shape: pl.BlockSpec(shape, lambda *_: (0,) * len(shape))
    hbm = pl.BlockSpec(memory_space=pl.ANY)
    return pl.pallas_call(
        _tc_body,
        in_specs=[
            full((B, D2)),
            full((8, D2)),
            full((D2, 2 * D2)),
            full((1, 2 * D2)),
            full((2 * D2, D2)),
            full((1, D2)),
            full((1, D2)),
            full((1, D2)),
            hbm,
            hbm,
            full((1, _G4)),
        ],
        out_specs=full((B, 1)),
        out_shape=jax.ShapeDtypeStruct((B, 1), jnp.float32),
        scratch_shapes=[
            pltpu.VMEM((_G4, D2), jnp.float32),
            pltpu.VMEM((_G4, HID), jnp.float32),
            pltpu.SemaphoreType.DMA,
            pltpu.SemaphoreType.DMA,
        ],
    )(q, s8, W1, b1, W2, b2, ln_g, ln_b, W_ih, W_hh, b4)


def kernel(query, support, symbol_emb, W1, b1, W2, b2, ln_g, ln_b, W_ih, W_hh, b_ih, b_hh):
    idx_q = query.reshape(-1).astype(jnp.int32)
    idx_s = jnp.concatenate([
        support.reshape(-1).astype(jnp.int32),
        jnp.zeros((_NS - 2 * FEW,), jnp.int32),
    ])
    rows_q, rows_s = _sc_gather(symbol_emb, idx_q, idx_s)
    q = rows_q.reshape(B, D2)
    s8 = rows_s.reshape(8, D2)

    bsum = b_ih + b_hh
    b4 = jnp.concatenate(
        [bsum[k * HID:k * HID + D2] for k in range(4)]).reshape(1, _G4)

    scores = _tc_dense(
        q, s8, W1, b1.reshape(1, -1), W2, b2.reshape(1, -1),
        ln_g.reshape(1, -1), ln_b.reshape(1, -1), W_ih, W_hh, b4)
    return scores.reshape(B)


# speedup vs baseline: 1.1094x; 1.1094x over previous
import functools

import jax
import jax.numpy as jnp
from jax import lax
from jax.experimental import pallas as pl
from jax.experimental.pallas import tpu as pltpu
from jax.experimental.pallas import tpu_sc as plsc

D = 128
D2 = 2 * D
HID = 2 * D2
H4 = 4 * HID
B = 1024
FEW = 5
STEPS = 4


_NW = 32
_RPW = B // _NW
_CH = 8
_NST = _RPW // _CH


def _sc_gather_body(table_hbm, idx_hbm, outq_hbm, outs_hbm,
                    idx_e, idx_o, out_v, idxs_v, outs_v, sem, sem_s):
    wid = lax.axis_index("s") * 2 + lax.axis_index("c")
    base = wid * _RPW
    pltpu.sync_copy(idx_hbm.at[pl.ds(base, _RPW)], idx_e)
    pltpu.sync_copy(idx_hbm.at[pl.ds(B + base, _RPW)], idx_o)
    copies = [
        pltpu.async_copy(
            table_hbm.at[idx_e.at[pl.ds(j * _CH, _CH)]],
            out_v.at[pl.ds(j * _CH, _CH), pl.ds(0, D)], sem)
        for j in range(_NST)
    ] + [
        pltpu.async_copy(
            table_hbm.at[idx_o.at[pl.ds(j * _CH, _CH)]],
            out_v.at[pl.ds(j * _CH, _CH), pl.ds(D, D)], sem)
        for j in range(_NST)
    ]

    @pl.when(wid == 0)
    def _():
        pltpu.sync_copy(idx_hbm.at[pl.ds(2 * B, 16)], idxs_v)
        cp_e = pltpu.async_copy(
            table_hbm.at[idxs_v.at[pl.ds(0, 8)]],
            outs_v.at[:, pl.ds(0, D)], sem_s)
        cp_o = pltpu.async_copy(
            table_hbm.at[idxs_v.at[pl.ds(8, 8)]],
            outs_v.at[:, pl.ds(D, D)], sem_s)
        cp_e.wait()
        cp_o.wait()
        pltpu.sync_copy(outs_v, outs_hbm)

    for c in copies:
        c.wait()
    pltpu.sync_copy(out_v, outq_hbm.at[pl.ds(base, _RPW)])


@functools.cache
def _make_sc_gather():
    return pl.kernel(
        _sc_gather_body,
        out_type=(
            jax.ShapeDtypeStruct((B, D2), jnp.float32),
            jax.ShapeDtypeStruct((8, D2), jnp.float32),
        ),
        mesh=plsc.VectorSubcoreMesh(core_axis_name="c", subcore_axis_name="s"),
        scratch_types=[
            pltpu.VMEM((_RPW,), jnp.int32),
            pltpu.VMEM((_RPW,), jnp.int32),
            pltpu.VMEM((_RPW, D2), jnp.float32),
            pltpu.VMEM((16,), jnp.int32),
            pltpu.VMEM((8, D2), jnp.float32),
            pltpu.SemaphoreType.DMA,
            pltpu.SemaphoreType.DMA,
        ],
    )


def _sc_gather(table, idx_flat):
    return _make_sc_gather()(table, idx_flat)


def _sigmoid(x):
    return 0.5 * jnp.tanh(0.5 * x) + 0.5


def _encode(x, W1, b1, W2, b2, ln_g, ln_b):
    h = jnp.maximum(jnp.dot(x, W1, preferred_element_type=jnp.float32) + b1, 0.0)
    h = jnp.dot(h, W2, preferred_element_type=jnp.float32) + b2
    y = h + x
    mu = jnp.mean(y, axis=-1, keepdims=True)
    var = jnp.mean((y - mu) * (y - mu), axis=-1, keepdims=True)
    return ln_g * (y - mu) * lax.rsqrt(var + 1e-5) + ln_b


def _dot_nt(x, w):
    return lax.dot_general(x, w, (((1,), (1,)), ((), ())),
                           preferred_element_type=jnp.float32)


_G4 = 4 * D2


def _tc_body(q_ref, s_ref, W1_ref, b1_ref, W2_ref, b2_ref, lng_ref, lnb_ref,
             Wih_hbm, Whh_hbm, bih_ref, bhh_ref, out_ref,
             wih_v, whh_v, sem_ih, sem_hh):
    cps_ih = [pltpu.async_copy(Wih_hbm.at[pl.ds(k * HID, D2)],
                               wih_v.at[pl.ds(k * D2, D2)], sem_ih)
              for k in range(4)]
    cps_hh = [pltpu.async_copy(Whh_hbm.at[pl.ds(k * HID, D2)],
                               whh_v.at[pl.ds(k * D2, D2)], sem_hh)
              for k in range(4)]

    W1 = W1_ref[...]
    b1 = b1_ref[...]
    W2 = W2_ref[...]
    b2 = b2_ref[...]
    ln_g = lng_ref[...]
    ln_b = lnb_ref[...]

    s_g = _encode(s_ref[...], W1, b1, W2, b2, ln_g, ln_b)
    row = lax.broadcasted_iota(jnp.int32, (8, 1), 0)
    s_g = jnp.where(row < FEW, s_g, 0.0)
    s_mean = jnp.sum(s_g, axis=0, keepdims=True) * (1.0 / FEW)

    q_g = _encode(q_ref[...], W1, b1, W2, b2, ln_g, ln_b)

    bsum = bih_ref[...] + bhh_ref[...]
    b4 = jnp.concatenate(
        [bsum[:, k * HID:k * HID + D2] for k in range(4)], axis=1)

    for cp in cps_ih:
        cp.wait()
    a = _dot_nt(q_g, wih_v[...]) + b4

    for cp in cps_hh:
        cp.wait()
    Whh_h = whh_v[:, :D2]
    Whh_r = whh_v[:, D2:]
    r_row = _dot_nt(s_mean, Whh_r)

    c = None
    h = None
    gates = a
    for step in range(STEPS):
        if step > 0:
            gates = a + r_row + _dot_nt(h, Whh_h)
        i = _sigmoid(gates[:, :D2])
        f = _sigmoid(gates[:, D2:2 * D2])
        g = jnp.tanh(gates[:, 2 * D2:3 * D2])
        o = _sigmoid(gates[:, 3 * D2:])
        c = f * c + i * g if step > 0 else i * g
        h = q_g + o * jnp.tanh(c)

    out_ref[...] = jnp.sum(h * s_mean, axis=1, keepdims=True)


@jax.jit
def _tc_dense(q, s, W1, b1, W2, b2, ln_g, ln_b, W_ih, W_hh, b_ih, b_hh):
    full = lambda shape: pl.BlockSpec(shape, lambda *_: (0,) * len(shape))
    hbm = pl.BlockSpec(memory_space=pl.ANY)
    return pl.pallas_call(
        _tc_body,
        in_specs=[
            full((B, D2)),
            full((8, D2)),
            full((D2, 2 * D2)),
            full((1, 2 * D2)),
            full((2 * D2, D2)),
            full((1, D2)),
            full((1, D2)),
            full((1, D2)),
            hbm,
            hbm,
            full((1, H4)),
            full((1, H4)),
        ],
        out_specs=full((B, 1)),
        out_shape=jax.ShapeDtypeStruct((B, 1), jnp.float32),
        scratch_shapes=[
            pltpu.VMEM((_G4, D2), jnp.float32),
            pltpu.VMEM((_G4, HID), jnp.float32),
            pltpu.SemaphoreType.DMA,
            pltpu.SemaphoreType.DMA,
        ],
    )(q, s, W1, b1, W2, b2, ln_g, ln_b, W_ih, W_hh, b_ih, b_hh)


def kernel(query, support, symbol_emb, W1, b1, W2, b2, ln_g, ln_b, W_ih, W_hh, b_ih, b_hh):
    if query.dtype != jnp.int32:
        query = query.astype(jnp.int32)
    if support.dtype != jnp.int32:
        support = support.astype(jnp.int32)
    pad3 = jnp.zeros((3,), jnp.int32)
    idx_flat = jnp.concatenate([
        query[:, 0], query[:, 1],
        support[:, 0], pad3, support[:, 1], pad3,
    ])
    q, s = _sc_gather(symbol_emb, idx_flat)

    scores = _tc_dense(
        q, s, W1, b1.reshape(1, -1), W2, b2.reshape(1, -1),
        ln_g.reshape(1, -1), ln_b.reshape(1, -1),
        W_ih, W_hh, b_ih.reshape(1, -1), b_hh.reshape(1, -1))
    return scores.reshape(B)
